# baseline (speedup 1.0000x reference)
import jax
import jax.numpy as jnp
from jax import lax
from jax.experimental import pallas as pl
from jax.experimental.pallas import tpu as pltpu
from jax.experimental.pallas import tpu_sc as plsc

N = 10000
E = 320000
D = 128
H = 256

GROUP = 128
NUM_GROUPS = E // GROUP
NC = 2
NS = 16
NW = NC * NS
GROUPS_PER_W = -(-NUM_GROUPS // NW)
STRIPE = 632
STRIDE = 624

NBUF = 2


def _stripe_start(sub):
    return pl.multiple_of(jnp.where(sub == NS - 1, N - STRIPE, sub * STRIDE), 8)


def _sc_scatter_kernel(edge_hbm, wm_hbm, zeros_hbm, out_hbm, *scratch):
    idx_all = scratch[0]
    rows_bufs = scratch[1:1 + NBUF]
    acc_shared = scratch[1 + NBUF]
    isem = scratch[2 + NBUF]
    lsems = scratch[3 + NBUF:3 + 2 * NBUF]
    ssems = scratch[3 + 2 * NBUF:3 + 3 * NBUF]

    core = lax.axis_index("core")
    sub = lax.axis_index("subcore")
    wid = sub * NC + core
    ni = jnp.where(wid < NUM_GROUPS % NW, GROUPS_PER_W, GROUPS_PER_W - 1)

    def group_off(i):
        return pl.multiple_of((i * NW + wid) * GROUP, GROUP)

    def start_load(i, k):
        pltpu.async_copy(edge_hbm.at[pl.ds(group_off(i), GROUP)],
                         rows_bufs[k], lsems[k])

    def wait_load(i, k):
        pltpu.make_async_copy(edge_hbm.at[pl.ds(group_off(i), GROUP)],
                              rows_bufs[k], lsems[k]).wait()

    def start_scatter(i, k):
        pltpu.async_copy(rows_bufs[k], acc_shared.at[idx_all.at[i]],
                         ssems[k], add=True)

    def wait_scatter(i, k):
        pltpu.make_async_copy(rows_bufs[k], acc_shared.at[idx_all.at[i]],
                              ssems[k]).wait()

    pltpu.async_copy(wm_hbm.at[wid], idx_all, isem)
    for k in range(NBUF):
        start_load(k, k)

    r0 = _stripe_start(sub)
    pltpu.sync_copy(zeros_hbm.at[pl.ds(r0, STRIPE)],
                    acc_shared.at[pl.ds(r0, STRIPE)])
    pltpu.make_async_copy(wm_hbm.at[wid], idx_all, isem).wait()
    plsc.subcore_barrier()

    @pl.loop(0, -(-GROUPS_PER_W // NBUF))
    def _(t):
        i0 = NBUF * t
        for k in range(NBUF):
            i = i0 + k

            @pl.when(i < ni)
            def _(i=i, k=k):
                wait_load(i, k)
                start_scatter(i, k)

        for k in range(NBUF):
            i = i0 + k

            @pl.when(i + NBUF < ni)
            def _(i=i, k=k):
                wait_scatter(i, k)
                start_load(i + NBUF, k)

    for k in range(NBUF):
        wait_scatter(0, k)

    plsc.subcore_barrier()
    pltpu.sync_copy(acc_shared.at[pl.ds(r0, STRIPE)],
                    out_hbm.at[core, pl.ds(r0, STRIPE), :])


def _sc_scatter(edge_attr, col_wm, zeros_nd):
    mesh = plsc.VectorSubcoreMesh(core_axis_name="core",
                                  subcore_axis_name="subcore")
    return pl.kernel(
        _sc_scatter_kernel,
        out_type=jax.ShapeDtypeStruct((NC, N, D), jnp.float32),
        mesh=mesh,
        scratch_types=(
            [pltpu.VMEM((GROUPS_PER_W, GROUP), jnp.int32)]
            + [pltpu.VMEM((GROUP, D), jnp.float32)] * NBUF
            + [pltpu.VMEM_SHARED((N, D), jnp.float32)]
            + [pltpu.SemaphoreType.DMA] * (1 + 2 * NBUF)
        ),
    )(edge_attr, col_wm, zeros_nd)


def _mlp_kernel(x_ref, p_ref, w1x_ref, w1a_ref, b1_ref, w2_ref, b2_ref, o_ref):
    agg = p_ref[0] + p_ref[1]
    h = jnp.dot(x_ref[...], w1x_ref[...], preferred_element_type=jnp.float32)
    h += jnp.dot(agg, w1a_ref[...], preferred_element_type=jnp.float32)
    h = jnp.maximum(h + b1_ref[...], 0.0)
    o_ref[...] = (jnp.dot(h, w2_ref[...], preferred_element_type=jnp.float32)
                  + b2_ref[...])


def _mlp(x, partials, W1, b1, W2, b2):
    R = 1000
    return pl.pallas_call(
        _mlp_kernel,
        grid=(N // R,),
        in_specs=[
            pl.BlockSpec((R, D), lambda i: (i, 0)),
            pl.BlockSpec((NC, R, D), lambda i: (0, i, 0)),
            pl.BlockSpec((D, H), lambda i: (0, 0)),
            pl.BlockSpec((D, H), lambda i: (0, 0)),
            pl.BlockSpec((1, H), lambda i: (0, 0)),
            pl.BlockSpec((H, D), lambda i: (0, 0)),
            pl.BlockSpec((1, D), lambda i: (0, 0)),
        ],
        out_specs=pl.BlockSpec((R, D), lambda i: (i, 0)),
        out_shape=jax.ShapeDtypeStruct((N, D), jnp.float32),
    )(x, partials, W1[:D], W1[D:], b1.reshape(1, H), W2, b2.reshape(1, D))


def kernel(x, edge_index, edge_attr, u, batch, W1, b1, W2, b2):
    col = edge_index[1]
    pad = NW * GROUPS_PER_W * GROUP - E
    col_wm = (jnp.concatenate([col, jnp.zeros((pad,), jnp.int32)])
              .reshape(GROUPS_PER_W, NW, GROUP)
              .transpose(1, 0, 2))
    zeros_nd = jnp.zeros((N, D), jnp.float32)
    partials = _sc_scatter(edge_attr, col_wm, zeros_nd)
    return _mlp(x, partials, W1, b1, W2, b2)


# speedup vs baseline: 1.2900x; 1.2900x over previous
import functools

import jax
import jax.numpy as jnp
from jax import lax
from jax.experimental import pallas as pl
from jax.experimental.pallas import tpu as pltpu
from jax.experimental.pallas import tpu_sc as plsc

N = 10000
E = 320000
D = 128
H = 256

GROUP = 128
NUM_GROUPS = E // GROUP
NC = 2
NS = 16
NW = NC * NS
GROUPS_PER_W = -(-NUM_GROUPS // NW)
STRIPE = 632
STRIDE = 624


def _stripe_start(sub):
    return pl.multiple_of(jnp.where(sub == NS - 1, N - STRIPE, sub * STRIDE), 8)


NBUF = 3


def _sc_scatter_kernel(edge_hbm, col_hbm, zeros_hbm, out_hbm, *scratch):
    idx_bufs = scratch[0:NBUF]
    rows_bufs = scratch[NBUF:2 * NBUF]
    acc_shared = scratch[2 * NBUF]
    lsems = scratch[2 * NBUF + 1:3 * NBUF + 1]
    ssems = scratch[3 * NBUF + 1:4 * NBUF + 1]

    core = lax.axis_index("core")
    sub = lax.axis_index("subcore")
    wid = sub * NC + core
    ni = jnp.where(wid < NUM_GROUPS % NW, GROUPS_PER_W, GROUPS_PER_W - 1)

    def group_off(i):
        return pl.multiple_of((i * NW + wid) * GROUP, GROUP)

    def start_loads(i, k):
        off = group_off(i)
        pltpu.async_copy(col_hbm.at[pl.ds(0, 2), pl.ds(off, GROUP)],
                         idx_bufs[k], lsems[k])
        pltpu.async_copy(edge_hbm.at[pl.ds(off, GROUP)], rows_bufs[k],
                         lsems[k])

    def wait_loads(i, k):
        off = group_off(i)
        pltpu.make_async_copy(col_hbm.at[pl.ds(0, 2), pl.ds(off, GROUP)],
                              idx_bufs[k], lsems[k]).wait()
        pltpu.make_async_copy(edge_hbm.at[pl.ds(off, GROUP)],
                              rows_bufs[k], lsems[k]).wait()

    def start_scatter(k):
        pltpu.async_copy(rows_bufs[k], acc_shared.at[idx_bufs[k].at[1]],
                         ssems[k], add=True)

    def wait_scatter(k):
        pltpu.make_async_copy(rows_bufs[k], acc_shared.at[idx_bufs[k].at[1]],
                              ssems[k]).wait()

    for k in range(NBUF):
        start_loads(k, k)

    r0 = _stripe_start(sub)
    pltpu.sync_copy(zeros_hbm.at[pl.ds(r0, STRIPE)],
                    acc_shared.at[pl.ds(r0, STRIPE)])
    plsc.subcore_barrier()

    @pl.loop(0, -(-GROUPS_PER_W // NBUF))
    def _(t):
        i0 = NBUF * t
        for k in range(NBUF):
            i = i0 + k

            @pl.when(i < ni)
            def _(i=i, k=k):
                wait_loads(i, k)
                start_scatter(k)

        for k in range(NBUF):
            i = i0 + k

            @pl.when(i + NBUF < ni)
            def _(i=i, k=k):
                wait_scatter(k)
                start_loads(i + NBUF, k)

    for k in range(NBUF):
        wait_scatter(k)

    plsc.subcore_barrier()
    pltpu.sync_copy(acc_shared.at[pl.ds(r0, STRIPE)],
                    out_hbm.at[core, pl.ds(r0, STRIPE), :])


def _sc_scatter(edge_attr, col2, zeros_nd):
    mesh = plsc.VectorSubcoreMesh(core_axis_name="core",
                                  subcore_axis_name="subcore")
    return pl.kernel(
        _sc_scatter_kernel,
        out_type=jax.ShapeDtypeStruct((NC, N, D), jnp.float32),
        mesh=mesh,
        scratch_types=(
            [pltpu.VMEM((2, GROUP), jnp.int32)] * NBUF
            + [pltpu.VMEM((GROUP, D), jnp.float32)] * NBUF
            + [pltpu.VMEM_SHARED((N, D), jnp.float32)]
            + [pltpu.SemaphoreType.DMA] * (2 * NBUF)
        ),
    )(edge_attr, col2, zeros_nd)


def _mlp_kernel(x_ref, p_ref, w1x_ref, w1a_ref, b1_ref, w2_ref, b2_ref, o_ref):
    agg = p_ref[0] + p_ref[1]
    h = jnp.dot(x_ref[...], w1x_ref[...], preferred_element_type=jnp.float32)
    h += jnp.dot(agg, w1a_ref[...], preferred_element_type=jnp.float32)
    h = jnp.maximum(h + b1_ref[...], 0.0)
    o_ref[...] = (jnp.dot(h, w2_ref[...], preferred_element_type=jnp.float32)
                  + b2_ref[...])


def _mlp(x, partials, W1, b1, W2, b2):
    R = 1000
    return pl.pallas_call(
        _mlp_kernel,
        grid=(N // R,),
        in_specs=[
            pl.BlockSpec((R, D), lambda i: (i, 0)),
            pl.BlockSpec((NC, R, D), lambda i: (0, i, 0)),
            pl.BlockSpec((D, H), lambda i: (0, 0)),
            pl.BlockSpec((D, H), lambda i: (0, 0)),
            pl.BlockSpec((1, H), lambda i: (0, 0)),
            pl.BlockSpec((H, D), lambda i: (0, 0)),
            pl.BlockSpec((1, D), lambda i: (0, 0)),
        ],
        out_specs=pl.BlockSpec((R, D), lambda i: (i, 0)),
        out_shape=jax.ShapeDtypeStruct((N, D), jnp.float32),
    )(x, partials, W1[:D], W1[D:], b1.reshape(1, H), W2, b2.reshape(1, D))


def kernel(x, edge_index, edge_attr, u, batch, W1, b1, W2, b2):
    zeros_nd = jnp.zeros((N, D), jnp.float32)
    partials = _sc_scatter(edge_attr, edge_index, zeros_nd)
    return _mlp(x, partials, W1, b1, W2, b2)


# speedup vs baseline: 1.3231x; 1.0257x over previous
import jax
import jax.numpy as jnp
from jax import lax
from jax.experimental import pallas as pl
from jax.experimental.pallas import tpu as pltpu
from jax.experimental.pallas import tpu_sc as plsc

N = 10000
E = 320000
D = 128
H = 256

GROUP = 128
NUM_GROUPS = E // GROUP
NC = 2
NS = 16
NW = NC * NS
GROUPS_PER_W = -(-NUM_GROUPS // NW)
STRIPE = 632
STRIDE = 624


def _stripe_start(sub):
    return pl.multiple_of(jnp.where(sub == NS - 1, N - STRIPE, sub * STRIDE), 8)


NBUF = 3


def _sc_scatter_kernel(edge_hbm, col_hbm, zeros_hbm, out_hbm, *scratch):
    idx_bufs = scratch[0:NBUF]
    rows_bufs = scratch[NBUF:2 * NBUF]
    acc_shared = scratch[2 * NBUF]
    lsems = scratch[2 * NBUF + 1:3 * NBUF + 1]
    ssems = scratch[3 * NBUF + 1:4 * NBUF + 1]

    core = lax.axis_index("core")
    sub = lax.axis_index("subcore")
    wid = sub * NC + core
    ni = jnp.where(wid < NUM_GROUPS % NW, GROUPS_PER_W, GROUPS_PER_W - 1)

    def group_off(i):
        return pl.multiple_of((i * NW + wid) * GROUP, GROUP)

    def start_loads(i, k):
        off = group_off(i)
        pltpu.async_copy(col_hbm.at[pl.ds(0, 2), pl.ds(off, GROUP)],
                         idx_bufs[k], lsems[k])
        pltpu.async_copy(edge_hbm.at[pl.ds(off, GROUP)], rows_bufs[k],
                         lsems[k])

    def wait_loads(i, k):
        off = group_off(i)
        pltpu.make_async_copy(col_hbm.at[pl.ds(0, 2), pl.ds(off, GROUP)],
                              idx_bufs[k], lsems[k]).wait()
        pltpu.make_async_copy(edge_hbm.at[pl.ds(off, GROUP)],
                              rows_bufs[k], lsems[k]).wait()

    def start_scatter(k):
        pltpu.async_copy(rows_bufs[k], acc_shared.at[idx_bufs[k].at[1]],
                         ssems[k], add=True)

    def wait_scatter(k):
        pltpu.make_async_copy(rows_bufs[k], acc_shared.at[idx_bufs[k].at[1]],
                              ssems[k]).wait()

    for k in range(NBUF):
        start_loads(k, k)

    r0 = _stripe_start(sub)
    pltpu.sync_copy(zeros_hbm.at[pl.ds(r0, STRIPE)],
                    acc_shared.at[pl.ds(r0, STRIPE)])
    plsc.subcore_barrier()

    @pl.loop(0, -(-GROUPS_PER_W // NBUF))
    def _(t):
        i0 = NBUF * t
        for k in range(NBUF):
            i = i0 + k

            @pl.when(i < ni)
            def _(i=i, k=k):
                wait_loads(i, k)
                start_scatter(k)

        for k in range(NBUF):
            i = i0 + k

            @pl.when(i + NBUF < ni)
            def _(i=i, k=k):
                wait_scatter(k)
                start_loads(i + NBUF, k)

    for k in range(NBUF):
        wait_scatter(k)

    plsc.subcore_barrier()
    pltpu.sync_copy(acc_shared.at[pl.ds(r0, STRIPE)],
                    out_hbm.at[core, pl.ds(r0, STRIPE), :])


def _sc_scatter(edge_attr, col2, zeros_nd):
    mesh = plsc.VectorSubcoreMesh(core_axis_name="core",
                                  subcore_axis_name="subcore")
    return pl.kernel(
        _sc_scatter_kernel,
        out_type=jax.ShapeDtypeStruct((NC, N, D), jnp.float32),
        mesh=mesh,
        scratch_types=(
            [pltpu.VMEM((2, GROUP), jnp.int32)] * NBUF
            + [pltpu.VMEM((GROUP, D), jnp.float32)] * NBUF
            + [pltpu.VMEM_SHARED((N, D), jnp.float32)]
            + [pltpu.SemaphoreType.DMA] * (2 * NBUF)
        ),
    )(edge_attr, col2, zeros_nd)


def _mlp_kernel(x_ref, p_ref, w1x_ref, w1a_ref, b1_ref, w2_ref, b2_ref, o_ref):
    agg = p_ref[0] + p_ref[1]
    h = jnp.dot(x_ref[...], w1x_ref[...], preferred_element_type=jnp.float32)
    h += jnp.dot(agg, w1a_ref[...], preferred_element_type=jnp.float32)
    h = jnp.maximum(h + b1_ref[...], 0.0)
    o_ref[...] = (jnp.dot(h, w2_ref[...], preferred_element_type=jnp.float32)
                  + b2_ref[...])


def _mlp(x, partials, W1, b1, W2, b2):
    R = 2000
    return pl.pallas_call(
        _mlp_kernel,
        grid=(N // R,),
        in_specs=[
            pl.BlockSpec((R, D), lambda i: (i, 0)),
            pl.BlockSpec((NC, R, D), lambda i: (0, i, 0)),
            pl.BlockSpec((D, H), lambda i: (0, 0)),
            pl.BlockSpec((D, H), lambda i: (0, 0)),
            pl.BlockSpec((1, H), lambda i: (0, 0)),
            pl.BlockSpec((H, D), lambda i: (0, 0)),
            pl.BlockSpec((1, D), lambda i: (0, 0)),
        ],
        out_specs=pl.BlockSpec((R, D), lambda i: (i, 0)),
        out_shape=jax.ShapeDtypeStruct((N, D), jnp.float32),
    )(x, partials, W1[:D], W1[D:], b1.reshape(1, H), W2, b2.reshape(1, D))


def kernel(x, edge_index, edge_attr, u, batch, W1, b1, W2, b2):
    zeros_nd = jnp.zeros((N, D), jnp.float32)
    partials = _sc_scatter(edge_attr, edge_index, zeros_nd)
    return _mlp(x, partials, W1, b1, W2, b2)
